# baseline (speedup 1.0000x reference)
import functools
import math

import jax
import jax.numpy as jnp
from jax import lax
from jax.experimental import pallas as pl
from jax.experimental.pallas import tpu as pltpu
from jax.experimental.pallas import tpu_sc as plsc

_EMB = 1024
_SEQ = 2048
_BATCH = 4
_ROWS = _SEQ * _BATCH
_NC, _NS = 2, 16
_NW = _NC * _NS
_RPW = _ROWS // _NW
_C = 8
_NCHUNK = _RPW // _C
_PC = _C // _BATCH
_NBUF = 12
_LANES = 16
_UNROLL = 4
_SCALE = math.sqrt(_EMB)


def _sc_embed(tok_flat, table, pos_embedding):
    mesh = plsc.VectorSubcoreMesh(core_axis_name="c", subcore_axis_name="s")

    @functools.partial(
        pl.kernel,
        out_type=jax.ShapeDtypeStruct((_SEQ, _BATCH, _EMB), jnp.float32),
        mesh=mesh,
        scratch_types=[
            pltpu.VMEM((_RPW,), jnp.int32),
            pltpu.VMEM((_NBUF, _C, _EMB), jnp.float32),
            pltpu.VMEM((_NBUF, _PC, 1, _EMB), jnp.float32),
            pltpu.SemaphoreType.DMA,
            pltpu.SemaphoreType.DMA,
            pltpu.SemaphoreType.DMA,
        ],
    )
    def k(tok_hbm, table_hbm, pe_hbm, out_hbm, idx_v, rows3, pos3, gsem, psem, osem):
        wid = lax.axis_index("s") * _NC + lax.axis_index("c")
        base = wid * _RPW
        pltpu.sync_copy(tok_hbm.at[pl.ds(pl.multiple_of(base, _RPW), _RPW)], idx_v)

        def issue(g, slot):
            ioff = pl.multiple_of(g * _C, _C)
            off = pl.multiple_of(base + g * _C, _C)
            pltpu.async_copy(
                table_hbm.at[idx_v.at[pl.ds(ioff, _C)]], rows3.at[slot], gsem)
            poff = pl.multiple_of(off // _BATCH, _PC)
            pltpu.async_copy(pe_hbm.at[pl.ds(poff, _PC)], pos3.at[slot], psem)

        for p in range(8):
            issue(p, p)

        def wait_out():
            for s in range(_PC):
                pltpu.make_async_copy(
                    rows3.at[0, pl.ds(0, _BATCH)], out_hbm.at[0], osem).wait()

        def chunk(g, carry):
            b = lax.rem(g, _NBUF)

            @pl.when(g + 8 < _NCHUNK)
            def _prefetch():
                @pl.when(g >= _NBUF - 8)
                def _drain_old_out():
                    wait_out()
                issue(g + 8, lax.rem(g + 8, _NBUF))

            pltpu.make_async_copy(
                table_hbm.at[idx_v.at[pl.ds(0, _C)]], rows3.at[b], gsem).wait()
            pltpu.make_async_copy(
                pe_hbm.at[pl.ds(0, _PC)], pos3.at[b], psem).wait()

            def quad(q, c2):
                def jstep(jj, c3):
                    for ju in range(_UNROLL):
                        sl = pl.ds((jj * _UNROLL + ju) * _LANES, _LANES)
                        pv = pos3[b, q, 0, sl]
                        for t in range(_BATCH):
                            r = q * _BATCH + t
                            rows3[b, r, sl] = rows3[b, r, sl] * _SCALE + pv
                    return c3

                return lax.fori_loop(0, _EMB // _LANES // _UNROLL, jstep, c2)

            lax.fori_loop(0, _PC, quad, 0)
            soff = (base + g * _C) // _BATCH
            for s in range(_PC):
                pltpu.async_copy(
                    rows3.at[b, pl.ds(s * _BATCH, _BATCH)],
                    out_hbm.at[soff + s], osem)
            return carry

        lax.fori_loop(0, _NCHUNK, chunk, 0)
        for _ in range(_NBUF):
            wait_out()

    return k(tok_flat, table, pos_embedding)


def kernel(tokens, table, pos_embedding):
    tok_flat = tokens.reshape(-1).astype(jnp.int32)
    return _sc_embed(tok_flat, table, pos_embedding)
